# baseline (speedup 1.0000x reference)
import functools

import jax
import jax.numpy as jnp
from jax import lax
from jax.experimental import pallas as pl
from jax.experimental.pallas import tpu as pltpu
from jax.experimental.pallas import tpu_sc as plsc


def _stage1(mem, put_idx, sample_idx):
    info = plsc.get_sparse_core_info()
    nc, ns, lanes = info.num_cores, info.num_subcores, info.num_lanes
    nw = nc * ns
    max_size = mem.shape[0]
    row = mem.shape[1]
    n_put = put_idx.shape[0]
    n_sample = sample_idx.shape[0]
    spt = n_sample // nw
    vpt = 1 << max(-(-max_size // nw) - 1, 1).bit_length()
    ver_total = vpt * nw
    chunk = 16384
    nchunk = n_put // chunk
    unroll = 4
    assert n_put % chunk == 0 and chunk % (lanes * unroll) == 0
    assert spt % lanes == 0

    mesh = plsc.VectorSubcoreMesh(core_axis_name="c", subcore_axis_name="s")

    @functools.partial(
        pl.kernel,
        mesh=mesh,
        out_type=(
            jax.ShapeDtypeStruct((ver_total,), jnp.int32),
            jax.ShapeDtypeStruct((n_sample, row), jnp.float32),
        ),
        scratch_types=[
            pltpu.VMEM((spt,), jnp.int32),
            pltpu.VMEM((chunk,), jnp.int32),
            pltpu.VMEM((chunk,), jnp.int32),
            pltpu.VMEM((vpt,), jnp.int32),
            pltpu.SemaphoreType.DMA,
            pltpu.SemaphoreType.DMA,
            pltpu.SemaphoreType.DMA,
        ],
        compiler_params=pltpu.CompilerParams(needs_layout_passes=False),
    )
    def k1(put_hbm, mem_hbm, sidx_hbm, ver_hbm, buf_hbm,
           sidxv, ch0, ch1, verv, semr, sem0, sem1):
        wid = lax.axis_index("s") * nc + lax.axis_index("c")
        base = wid * spt
        lo = wid * vpt
        zero16 = jnp.zeros((lanes,), jnp.int32)
        iota1 = jnp.arange(lanes, dtype=jnp.int32) + 1

        pltpu.sync_copy(sidx_hbm.at[pl.ds(base, spt)], sidxv)

        def fire(k, _):
            v = sidxv[pl.ds(k * lanes, lanes)]
            for u in range(lanes):
                s = k * lanes + u
                pltpu.async_copy(
                    mem_hbm.at[v[u]], buf_hbm.at[base + s], semr)
            return 0

        lax.fori_loop(0, spt // lanes, fire, 0)

        bufs = (ch0, ch1)
        sems = (sem0, sem1)
        copies = [None, None]
        copies[0] = pltpu.async_copy(put_hbm.at[pl.ds(0, chunk)], ch0, sem0)

        def zbody(i, _):
            zb = i * (lanes * 8)
            for u in range(8):
                verv[pl.ds(zb + u * lanes, lanes)] = zero16
            return 0

        lax.fori_loop(0, vpt // (lanes * 8), zbody, 0)

        for c in range(nchunk):
            if c + 1 < nchunk:
                copies[(c + 1) % 2] = pltpu.async_copy(
                    put_hbm.at[pl.ds((c + 1) * chunk, chunk)],
                    bufs[(c + 1) % 2], sems[(c + 1) % 2])
            copies[c % 2].wait()
            ch = bufs[c % 2]

            def vbody(k, _, _c=c, _ch=ch):
                vb = k * (lanes * unroll)
                for u in range(unroll):
                    off = vb + u * lanes
                    idx = _ch[pl.ds(off, lanes)]
                    loc = idx - lo
                    m = loc.astype(jnp.uint32) < jnp.uint32(vpt)
                    locc = loc & (vpt - 1)
                    jv = iota1 + (_c * chunk + off)
                    plsc.store_scatter(verv, [locc], jv, mask=m)
                return 0

            lax.fori_loop(0, chunk // (lanes * unroll), vbody, 0)

        pltpu.sync_copy(verv, ver_hbm.at[pl.ds(lo, vpt)])

        def drain(k, _):
            for u in range(lanes):
                s = k * lanes + u
                pltpu.make_async_copy(
                    mem_hbm.at[0], buf_hbm.at[base + s], semr).wait()
            return 0

        lax.fori_loop(0, spt // lanes, drain, 0)

    return k1(put_idx, mem, sample_idx)


def _stage2(buf, ver, put_val, sample_idx):
    info = plsc.get_sparse_core_info()
    nc, ns, lanes = info.num_cores, info.num_subcores, info.num_lanes
    nw = nc * ns
    n_sample = sample_idx.shape[0]
    row = put_val.shape[1]
    spt = n_sample // nw
    assert spt % lanes == 0

    mesh = plsc.VectorSubcoreMesh(core_axis_name="c", subcore_axis_name="s")

    @functools.partial(
        pl.kernel,
        mesh=mesh,
        out_type=tuple(
            jax.ShapeDtypeStruct((n_sample,), jnp.float32) for _ in range(8)),
        scratch_types=[
            pltpu.VMEM((spt,), jnp.int32),
            pltpu.VMEM((spt,), jnp.int32),
            pltpu.VMEM((spt,), jnp.int32),
            pltpu.VMEM((spt, 8), jnp.float32),
            pltpu.VMEM((spt, 8), jnp.float32),
            pltpu.VMEM((8, spt), jnp.float32),
            pltpu.SemaphoreType.DMA,
            pltpu.SemaphoreType.DMA,
        ],
        compiler_params=pltpu.CompilerParams(
            needs_layout_passes=False, use_tc_tiling_on_sc=False),
    )
    def k2(buf_hbm, ver_hbm, pval_hbm, sidx_hbm,
           o0, o1, o2, o3, o4, o5, o6, o7,
           sidxv, vv, pv, rowsv, pvalv, selv, sem1, sem2):
        wid = lax.axis_index("s") * nc + lax.axis_index("c")
        base = wid * spt
        outs = (o0, o1, o2, o3, o4, o5, o6, o7)
        pltpu.sync_copy(sidx_hbm.at[pl.ds(base, spt)], sidxv)
        cp_rows = pltpu.async_copy(
            buf_hbm.at[pl.ds(base, spt)], rowsv, sem1)
        cp_ver = pltpu.async_copy(ver_hbm.at[sidxv], vv, sem2)
        cp_ver.wait()

        def pbody(k, _):
            v = vv[pl.ds(k * lanes, lanes)]
            pv[pl.ds(k * lanes, lanes)] = jnp.maximum(v - 1, 0)
            return 0

        lax.fori_loop(0, spt // lanes, pbody, 0)
        cp_pval = pltpu.async_copy(pval_hbm.at[pv], pvalv, sem2)
        cp_rows.wait()
        cp_pval.wait()

        iota = jnp.arange(lanes, dtype=jnp.int32)

        def sbody(k, _):
            vvv = vv[pl.ds(k * lanes, lanes)]
            m = vvv > 0
            rvec = iota + k * lanes
            for co in range(8):
                cosplat = jnp.full((lanes,), co, jnp.int32)
                mval = plsc.load_gather(rowsv, [rvec, cosplat])
                pval = plsc.load_gather(pvalv, [rvec, cosplat])
                sel = jnp.where(m, pval, mval)
                selv[co, pl.ds(k * lanes, lanes)] = sel
            return 0

        lax.fori_loop(0, spt // lanes, sbody, 0)

        for co in range(8):
            pltpu.sync_copy(selv.at[co], outs[co].at[pl.ds(base, spt)])

    return k2(buf, ver, put_val, sample_idx)


def kernel(mem, put_idx, put_val, sample_idx):
    put_idx = put_idx.astype(jnp.int32)
    sample_idx = sample_idx.astype(jnp.int32)
    ver, buf = _stage1(mem, put_idx, sample_idx)
    cols = _stage2(buf, ver, put_val, sample_idx)
    state = jnp.stack(cols[0:3], axis=1)
    action = cols[3].reshape(-1, 1)
    reward = cols[4].reshape(-1, 1)
    next_state = jnp.stack(cols[5:8], axis=1)
    return (state, action, reward, next_state)


# speedup vs baseline: 1.5160x; 1.5160x over previous
import functools

import jax
import jax.numpy as jnp
from jax import lax
from jax.experimental import pallas as pl
from jax.experimental.pallas import tpu as pltpu
from jax.experimental.pallas import tpu_sc as plsc


def _build_ver(put_idx, max_size):
    info = plsc.get_sparse_core_info()
    nc, ns, lanes = info.num_cores, info.num_subcores, info.num_lanes
    nw = nc * ns
    n_put = put_idx.shape[0]
    vpt = 1 << max(-(-max_size // nw) - 1, 1).bit_length()
    ver_total = vpt * nw
    chunk = 16384
    nchunk = n_put // chunk
    unroll = 4
    assert n_put % chunk == 0 and chunk % (lanes * unroll) == 0

    mesh = plsc.VectorSubcoreMesh(core_axis_name="c", subcore_axis_name="s")

    @functools.partial(
        pl.kernel,
        mesh=mesh,
        out_type=jax.ShapeDtypeStruct((ver_total,), jnp.int32),
        scratch_types=[
            pltpu.VMEM((chunk,), jnp.int32),
            pltpu.VMEM((chunk,), jnp.int32),
            pltpu.VMEM((vpt,), jnp.int32),
            pltpu.SemaphoreType.DMA,
            pltpu.SemaphoreType.DMA,
        ],
        compiler_params=pltpu.CompilerParams(needs_layout_passes=False),
    )
    def ka(put_hbm, ver_hbm, ch0, ch1, verv, sem0, sem1):
        wid = lax.axis_index("s") * nc + lax.axis_index("c")
        lo = wid * vpt
        zero16 = jnp.zeros((lanes,), jnp.int32)
        iota1 = jnp.arange(lanes, dtype=jnp.int32) + 1

        bufs = (ch0, ch1)
        sems = (sem0, sem1)
        copies = [None, None]
        copies[0] = pltpu.async_copy(put_hbm.at[pl.ds(0, chunk)], ch0, sem0)

        def zbody(i, _):
            zb = i * (lanes * 8)
            for u in range(8):
                verv[pl.ds(zb + u * lanes, lanes)] = zero16
            return 0

        lax.fori_loop(0, vpt // (lanes * 8), zbody, 0)

        for c in range(nchunk):
            if c + 1 < nchunk:
                copies[(c + 1) % 2] = pltpu.async_copy(
                    put_hbm.at[pl.ds((c + 1) * chunk, chunk)],
                    bufs[(c + 1) % 2], sems[(c + 1) % 2])
            copies[c % 2].wait()
            ch = bufs[c % 2]

            def vbody(k, _, _c=c, _ch=ch):
                vb = k * (lanes * unroll)
                for u in range(unroll):
                    off = vb + u * lanes
                    idx = _ch[pl.ds(off, lanes)]
                    loc = idx - lo
                    m = loc.astype(jnp.uint32) < jnp.uint32(vpt)
                    locc = loc & (vpt - 1)
                    jv = iota1 + (_c * chunk + off)
                    plsc.store_scatter(verv, [locc], jv, mask=m)
                return 0

            lax.fori_loop(0, chunk // (lanes * unroll), vbody, 0)

        pltpu.sync_copy(verv, ver_hbm.at[pl.ds(lo, vpt)])

    return ka(put_idx)


def _sample(mem, put_val, sample_idx, ver2d):
    info = plsc.get_sparse_core_info()
    nc, ns, lanes = info.num_cores, info.num_subcores, info.num_lanes
    nw = nc * ns
    n_sample = sample_idx.shape[0]
    spt = n_sample // nw
    assert spt % lanes == 0

    mesh = plsc.VectorSubcoreMesh(core_axis_name="c", subcore_axis_name="s")

    @functools.partial(
        pl.kernel,
        mesh=mesh,
        out_type=tuple(
            jax.ShapeDtypeStruct((n_sample,), jnp.float32) for _ in range(8)),
        scratch_types=[
            pltpu.VMEM((spt,), jnp.int32),
            pltpu.VMEM((spt,), jnp.int32),
            pltpu.VMEM((spt, 8), jnp.int32),
            pltpu.VMEM((spt,), jnp.int32),
            pltpu.VMEM((spt,), jnp.int32),
            pltpu.VMEM((spt, 8), jnp.float32),
            pltpu.VMEM((spt, 8), jnp.float32),
            pltpu.VMEM((8, spt), jnp.float32),
            pltpu.SemaphoreType.DMA,
            pltpu.SemaphoreType.DMA,
        ],
        compiler_params=pltpu.CompilerParams(
            needs_layout_passes=False, use_tc_tiling_on_sc=False),
    )
    def kb(mem_hbm, pval_hbm, sidx_hbm, ver_hbm,
           o0, o1, o2, o3, o4, o5, o6, o7,
           sidxv, sg, vrows, vv, pv, rowsv, pvalv, selv, sem1, sem2):
        wid = lax.axis_index("s") * nc + lax.axis_index("c")
        base = wid * spt
        outs = (o0, o1, o2, o3, o4, o5, o6, o7)
        pltpu.sync_copy(sidx_hbm.at[pl.ds(base, spt)], sidxv)
        cp_rows = pltpu.async_copy(mem_hbm.at[sidxv], rowsv, sem1)

        def gbody(k, _):
            s = sidxv[pl.ds(k * lanes, lanes)]
            sg[pl.ds(k * lanes, lanes)] = s >> 3
            return 0

        lax.fori_loop(0, spt // lanes, gbody, 0)
        cp_ver = pltpu.async_copy(ver_hbm.at[sg], vrows, sem2)
        cp_ver.wait()

        iota = jnp.arange(lanes, dtype=jnp.int32)

        def pbody(k, _):
            s = sidxv[pl.ds(k * lanes, lanes)]
            rvec = iota + k * lanes
            v = plsc.load_gather(vrows, [rvec, s & 7])
            vv[pl.ds(k * lanes, lanes)] = v
            pv[pl.ds(k * lanes, lanes)] = jnp.maximum(v - 1, 0)
            return 0

        lax.fori_loop(0, spt // lanes, pbody, 0)
        cp_pval = pltpu.async_copy(pval_hbm.at[pv], pvalv, sem2)
        cp_rows.wait()
        cp_pval.wait()

        def sbody(k, _):
            vvv = vv[pl.ds(k * lanes, lanes)]
            m = vvv > 0
            rvec = iota + k * lanes
            for co in range(8):
                cosplat = jnp.full((lanes,), co, jnp.int32)
                mval = plsc.load_gather(rowsv, [rvec, cosplat])
                pval = plsc.load_gather(pvalv, [rvec, cosplat])
                sel = jnp.where(m, pval, mval)
                selv[co, pl.ds(k * lanes, lanes)] = sel
            return 0

        lax.fori_loop(0, spt // lanes, sbody, 0)

        for co in range(8):
            pltpu.sync_copy(selv.at[co], outs[co].at[pl.ds(base, spt)])

    return kb(mem, put_val, sample_idx, ver2d)


def kernel(mem, put_idx, put_val, sample_idx):
    put_idx = put_idx.astype(jnp.int32)
    sample_idx = sample_idx.astype(jnp.int32)
    ver = _build_ver(put_idx, mem.shape[0])
    ver2d = ver.reshape(-1, 8)
    cols = _sample(mem, put_val, sample_idx, ver2d)
    state = jnp.stack(cols[0:3], axis=1)
    action = cols[3].reshape(-1, 1)
    reward = cols[4].reshape(-1, 1)
    next_state = jnp.stack(cols[5:8], axis=1)
    return (state, action, reward, next_state)
